# Initial kernel scaffold; baseline (speedup 1.0000x reference)
import jax
import jax.numpy as jnp
from jax.experimental import pallas as pl


def kernel(state, w1p, w2p, whp):
    raise NotImplementedError("write your pallas kernel here")


# speedup vs baseline: 1.4157x; 1.4157x over previous
import functools

import jax
import jax.numpy as jnp
from jax.experimental import pallas as pl
from jax.experimental.pallas import tpu as pltpu


def _round_up(x, m):
    return ((x + m - 1) // m) * m


def _actor_kernel(s_ref, w1p_ref, w2p_ref, whp_ref, mu_ref, ls_ref):
    n_in = w1p_ref.shape[0] - 1
    n_hid = w2p_ref.shape[0] - 1
    n_act = mu_ref.shape[1]

    x = s_ref[...].astype(jnp.bfloat16)

    h = jnp.dot(x, w1p_ref[:n_in, :], preferred_element_type=jnp.float32)
    h = h + w1p_ref[n_in:n_in + 1, :].astype(jnp.float32)
    h = jnp.maximum(h, 0.0).astype(jnp.bfloat16)

    h = jnp.dot(h, w2p_ref[:n_hid, :], preferred_element_type=jnp.float32)
    h = h + w2p_ref[n_hid:n_hid + 1, :].astype(jnp.float32)
    h = jnp.maximum(h, 0.0).astype(jnp.bfloat16)

    head = jnp.dot(h, whp_ref[:n_hid, :], preferred_element_type=jnp.float32)
    head = head + whp_ref[n_hid:n_hid + 1, :].astype(jnp.float32)

    mu_ref[...] = head[:, :n_act]
    ls_ref[...] = jnp.clip(head[:, n_act:], -20.0, 2.0)


@functools.partial(jax.jit, static_argnames=("tm_max",))
def _actor_forward(state, w1p, w2p, whp, *, tm_max=1024):
    B, n_in = state.shape
    n_hid = w2p.shape[0] - 1
    n_act2 = whp.shape[1]
    n_act = n_act2 // 2

    tm = min(tm_max, _round_up(B, 8))
    b_pad = _round_up(B, tm)
    if b_pad != B:
        state = jnp.pad(state, ((0, b_pad - B), (0, 0)))
    grid = (b_pad // tm,)

    w1b = w1p.astype(jnp.bfloat16)
    w2b = w2p.astype(jnp.bfloat16)
    whb = whp.astype(jnp.bfloat16)

    flops = 2 * b_pad * (n_in * n_hid + n_hid * n_hid + n_hid * n_act2)
    bytes_accessed = 4 * (b_pad * n_in + b_pad * n_act2) + 2 * (
        w1b.size + w2b.size + whb.size)

    mu, ls = pl.pallas_call(
        _actor_kernel,
        out_shape=(
            jax.ShapeDtypeStruct((b_pad, n_act), jnp.float32),
            jax.ShapeDtypeStruct((b_pad, n_act), jnp.float32),
        ),
        grid=grid,
        in_specs=[
            pl.BlockSpec((tm, n_in), lambda i: (i, 0)),
            pl.BlockSpec((n_in + 1, n_hid), lambda i: (0, 0)),
            pl.BlockSpec((n_hid + 1, n_hid), lambda i: (0, 0)),
            pl.BlockSpec((n_hid + 1, n_act2), lambda i: (0, 0)),
        ],
        out_specs=(
            pl.BlockSpec((tm, n_act), lambda i: (i, 0)),
            pl.BlockSpec((tm, n_act), lambda i: (i, 0)),
        ),
        compiler_params=pltpu.CompilerParams(
            dimension_semantics=("parallel",)),
        cost_estimate=pl.CostEstimate(
            flops=flops, transcendentals=0, bytes_accessed=bytes_accessed),
    )(state, w1b, w2b, whb)

    return mu[:B], ls[:B]


def kernel(state, w1p, w2p, whp):
    return _actor_forward(state, w1p, w2p, whp, tm_max=1024)


# speedup vs baseline: 1.5806x; 1.1165x over previous
import functools

import jax
import jax.numpy as jnp
from jax.experimental import pallas as pl
from jax.experimental.pallas import tpu as pltpu


def _round_up(x, m):
    return ((x + m - 1) // m) * m


def _actor_kernel(s_ref, w1p_ref, w2p_ref, whp_ref, mu_ref, ls_ref):
    n_in = w1p_ref.shape[0] - 1
    n_hid = w2p_ref.shape[0] - 1
    n_act = mu_ref.shape[1]

    x = s_ref[...].astype(jnp.bfloat16)

    h = jnp.dot(x, w1p_ref[:n_in, :], preferred_element_type=jnp.float32)
    h = h + w1p_ref[n_in:n_in + 1, :].astype(jnp.float32)
    h = jnp.maximum(h, 0.0).astype(jnp.bfloat16)

    h = jnp.dot(h, w2p_ref[:n_hid, :], preferred_element_type=jnp.float32)
    h = h + w2p_ref[n_hid:n_hid + 1, :].astype(jnp.float32)
    h = jnp.maximum(h, 0.0).astype(jnp.bfloat16)

    head = jnp.dot(h, whp_ref[:n_hid, :], preferred_element_type=jnp.float32)
    head = head + whp_ref[n_hid:n_hid + 1, :].astype(jnp.float32)

    mu_ref[...] = head[:, :n_act]
    ls_ref[...] = jnp.clip(head[:, n_act:], -20.0, 2.0)


@functools.partial(jax.jit, static_argnames=("tm_max",))
def _actor_forward(state, w1p, w2p, whp, *, tm_max=1024):
    B, n_in = state.shape
    n_hid = w2p.shape[0] - 1
    n_act2 = whp.shape[1]
    n_act = n_act2 // 2

    tm = min(tm_max, _round_up(B, 8))
    b_pad = _round_up(B, tm)
    if b_pad != B:
        state = jnp.pad(state, ((0, b_pad - B), (0, 0)))
    grid = (b_pad // tm,)

    w1b = w1p.astype(jnp.bfloat16)
    w2b = w2p.astype(jnp.bfloat16)
    whb = whp.astype(jnp.bfloat16)

    flops = 2 * b_pad * (n_in * n_hid + n_hid * n_hid + n_hid * n_act2)
    bytes_accessed = 4 * (b_pad * n_in + b_pad * n_act2) + 2 * (
        w1b.size + w2b.size + whb.size)

    mu, ls = pl.pallas_call(
        _actor_kernel,
        out_shape=(
            jax.ShapeDtypeStruct((b_pad, n_act), jnp.float32),
            jax.ShapeDtypeStruct((b_pad, n_act), jnp.float32),
        ),
        grid=grid,
        in_specs=[
            pl.BlockSpec((tm, n_in), lambda i: (i, 0)),
            pl.BlockSpec((n_in + 1, n_hid), lambda i: (0, 0)),
            pl.BlockSpec((n_hid + 1, n_hid), lambda i: (0, 0)),
            pl.BlockSpec((n_hid + 1, n_act2), lambda i: (0, 0)),
        ],
        out_specs=(
            pl.BlockSpec((tm, n_act), lambda i: (i, 0)),
            pl.BlockSpec((tm, n_act), lambda i: (i, 0)),
        ),
        compiler_params=pltpu.CompilerParams(
            dimension_semantics=("parallel",)),
        cost_estimate=pl.CostEstimate(
            flops=flops, transcendentals=0, bytes_accessed=bytes_accessed),
    )(state, w1b, w2b, whb)

    return mu[:B], ls[:B]


def kernel(state, w1p, w2p, whp):
    return _actor_forward(state, w1p, w2p, whp, tm_max=2048)


# speedup vs baseline: 1.5943x; 1.0086x over previous
import functools

import jax
import jax.numpy as jnp
from jax.experimental import pallas as pl
from jax.experimental.pallas import tpu as pltpu


def _round_up(x, m):
    return ((x + m - 1) // m) * m


def _actor_kernel(s_ref, w1p_ref, w2p_ref, whp_ref, mu_ref, ls_ref):
    n_in = w1p_ref.shape[0] - 1
    n_hid = w2p_ref.shape[0] - 1
    n_act = mu_ref.shape[1]

    x = s_ref[...].astype(jnp.bfloat16)

    h = jnp.dot(x, w1p_ref[:n_in, :], preferred_element_type=jnp.float32)
    h = h + w1p_ref[n_in:n_in + 1, :].astype(jnp.float32)
    h = jnp.maximum(h, 0.0).astype(jnp.bfloat16)

    h = jnp.dot(h, w2p_ref[:n_hid, :], preferred_element_type=jnp.float32)
    h = h + w2p_ref[n_hid:n_hid + 1, :].astype(jnp.float32)
    h = jnp.maximum(h, 0.0).astype(jnp.bfloat16)

    head = jnp.dot(h, whp_ref[:n_hid, :], preferred_element_type=jnp.float32)
    head = head + whp_ref[n_hid:n_hid + 1, :].astype(jnp.float32)

    mu_ref[...] = head[:, :n_act]
    ls_ref[...] = jnp.clip(head[:, n_act:], -20.0, 2.0)


@functools.partial(jax.jit, static_argnames=("tm_max",))
def _actor_forward(state, w1p, w2p, whp, *, tm_max=1024):
    B, n_in = state.shape
    n_hid = w2p.shape[0] - 1
    n_act2 = whp.shape[1]
    n_act = n_act2 // 2

    tm = min(tm_max, _round_up(B, 8))
    b_pad = _round_up(B, tm)
    if b_pad != B:
        state = jnp.pad(state, ((0, b_pad - B), (0, 0)))
    grid = (b_pad // tm,)

    w1b = w1p.astype(jnp.bfloat16)
    w2b = w2p.astype(jnp.bfloat16)
    whb = whp.astype(jnp.bfloat16)

    flops = 2 * b_pad * (n_in * n_hid + n_hid * n_hid + n_hid * n_act2)
    bytes_accessed = 4 * (b_pad * n_in + b_pad * n_act2) + 2 * (
        w1b.size + w2b.size + whb.size)

    mu, ls = pl.pallas_call(
        _actor_kernel,
        out_shape=(
            jax.ShapeDtypeStruct((b_pad, n_act), jnp.float32),
            jax.ShapeDtypeStruct((b_pad, n_act), jnp.float32),
        ),
        grid=grid,
        in_specs=[
            pl.BlockSpec((tm, n_in), lambda i: (i, 0)),
            pl.BlockSpec((n_in + 1, n_hid), lambda i: (0, 0)),
            pl.BlockSpec((n_hid + 1, n_hid), lambda i: (0, 0)),
            pl.BlockSpec((n_hid + 1, n_act2), lambda i: (0, 0)),
        ],
        out_specs=(
            pl.BlockSpec((tm, n_act), lambda i: (i, 0)),
            pl.BlockSpec((tm, n_act), lambda i: (i, 0)),
        ),
        compiler_params=pltpu.CompilerParams(
            dimension_semantics=("parallel",)),
        cost_estimate=pl.CostEstimate(
            flops=flops, transcendentals=0, bytes_accessed=bytes_accessed),
    )(state, w1b, w2b, whb)

    return mu[:B], ls[:B]


def kernel(state, w1p, w2p, whp):
    return _actor_forward(state, w1p, w2p, whp, tm_max=4096)


# speedup vs baseline: 1.9418x; 1.2180x over previous
import functools

import jax
import jax.numpy as jnp
from jax.experimental import pallas as pl
from jax.experimental.pallas import tpu as pltpu


def _round_up(x, m):
    return ((x + m - 1) // m) * m


def _actor_kernel(s_ref, w1p_ref, w2p_ref, whp_ref, mu_ref, ls_ref):
    n_in = w1p_ref.shape[0] - 1
    n_hid = w2p_ref.shape[0] - 1
    n_act = mu_ref.shape[1]

    x = s_ref[...].astype(jnp.bfloat16)

    h = jnp.dot(x, w1p_ref[:n_in, :].astype(jnp.bfloat16),
                preferred_element_type=jnp.float32)
    h = h + w1p_ref[n_in:n_in + 1, :]
    h = jnp.maximum(h, 0.0).astype(jnp.bfloat16)

    h = jnp.dot(h, w2p_ref[:n_hid, :].astype(jnp.bfloat16),
                preferred_element_type=jnp.float32)
    h = h + w2p_ref[n_hid:n_hid + 1, :]
    h = jnp.maximum(h, 0.0).astype(jnp.bfloat16)

    head = jnp.dot(h, whp_ref[:n_hid, :].astype(jnp.bfloat16),
                   preferred_element_type=jnp.float32)
    head = head + whp_ref[n_hid:n_hid + 1, :]

    mu_ref[...] = head[:, :n_act]
    ls_ref[...] = jnp.clip(head[:, n_act:], -20.0, 2.0)


@functools.partial(jax.jit, static_argnames=("tm_max",))
def _actor_forward(state, w1p, w2p, whp, *, tm_max=1024):
    B, n_in = state.shape
    n_hid = w2p.shape[0] - 1
    n_act2 = whp.shape[1]
    n_act = n_act2 // 2

    tm = min(tm_max, _round_up(B, 8))
    b_pad = _round_up(B, tm)
    if b_pad != B:
        state = jnp.pad(state, ((0, b_pad - B), (0, 0)))
    grid = (b_pad // tm,)

    flops = 2 * b_pad * (n_in * n_hid + n_hid * n_hid + n_hid * n_act2)
    bytes_accessed = 4 * (b_pad * n_in + b_pad * n_act2
                          + w1p.size + w2p.size + whp.size)

    mu, ls = pl.pallas_call(
        _actor_kernel,
        out_shape=(
            jax.ShapeDtypeStruct((b_pad, n_act), jnp.float32),
            jax.ShapeDtypeStruct((b_pad, n_act), jnp.float32),
        ),
        grid=grid,
        in_specs=[
            pl.BlockSpec((tm, n_in), lambda i: (i, 0)),
            pl.BlockSpec((n_in + 1, n_hid), lambda i: (0, 0)),
            pl.BlockSpec((n_hid + 1, n_hid), lambda i: (0, 0)),
            pl.BlockSpec((n_hid + 1, n_act2), lambda i: (0, 0)),
        ],
        out_specs=(
            pl.BlockSpec((tm, n_act), lambda i: (i, 0)),
            pl.BlockSpec((tm, n_act), lambda i: (i, 0)),
        ),
        compiler_params=pltpu.CompilerParams(
            dimension_semantics=("parallel",)),
        cost_estimate=pl.CostEstimate(
            flops=flops, transcendentals=0, bytes_accessed=bytes_accessed),
    )(state, w1p, w2p, whp)

    return mu[:B], ls[:B]


def kernel(state, w1p, w2p, whp):
    return _actor_forward(state, w1p, w2p, whp, tm_max=4096)
